# baseline (speedup 1.0000x reference)
import jax
import jax.numpy as jnp
from jax import lax
from jax.experimental import pallas as pl
from jax.experimental.pallas import tpu as pltpu
from jax.experimental.pallas import tpu_sc as plsc

N = 10000
E = 320000
D_IN = 128
D_HID = 16
D_OUT = 128

NC = 2
NS = 16
NW = NC * NS
B = 128
GTOT = E // B
K = 79
KB = K * B
NPAD = 10240
RPC = NPAD // NS
DEPTH = 4
GD = 8

_sc_mesh = plsc.VectorSubcoreMesh(
    core_axis_name="c", subcore_axis_name="s", num_cores=NC, num_subcores=NS)
_sc_params = pltpu.CompilerParams(use_tc_tiling_on_sc=False)

_f32 = jnp.float32


def _worker_span(c, s):
  wid = s * NC + c
  gbase = jnp.minimum(wid * K, GTOT - K)
  j0 = wid * K - gbase
  return gbase, j0


def _fill_rows(buf, n, val):
  def st(i, carry):
    buf[i] = jnp.full((D_HID,), val, _f32)
    return carry
  lax.fori_loop(0, n, st, 0)


def _fill_flat(buf, n, val):
  def st(i, carry):
    buf[pl.ds(i * D_HID, D_HID)] = jnp.full((D_HID,), val, _f32)
    return carry
  lax.fori_loop(0, n // D_HID, st, 0)


def _deg_body(ei_hbm, out_hbm, acc_sh, dst_1d, ones_v, zbuf, sem0, sem1):
  c = lax.axis_index("c")
  s = lax.axis_index("s")
  gbase, j0 = _worker_span(c, s)
  rows = pl.ds(s * RPC, RPC)
  pltpu.async_copy(ei_hbm.at[1, pl.ds(gbase * B, KB)], dst_1d, sem1)
  _fill_flat(zbuf, RPC, 0.0)
  _fill_flat(ones_v, B, 1.0)
  pltpu.sync_copy(zbuf, acc_sh.at[rows])
  pltpu.make_async_copy(ei_hbm.at[1, pl.ds(0, KB)], dst_1d, sem1).wait()
  plsc.subcore_barrier()

  def issue(j):
    pltpu.async_copy(ones_v, acc_sh.at[dst_1d.at[pl.ds(j * B, B)]], sem0,
                     add=True)

  def wait_one():
    pltpu.make_async_copy(ones_v, acc_sh.at[dst_1d.at[pl.ds(0, B)]],
                          sem0).wait()

  def prime(j, carry):
    issue(j0 + j)
    return carry
  lax.fori_loop(0, DEPTH, prime, 0)

  def step(j, carry):
    wait_one()
    issue(j)
    return carry
  lax.fori_loop(j0 + DEPTH, K, step, 0)

  def drain(j, carry):
    wait_one()
    return carry
  lax.fori_loop(0, DEPTH, drain, 0)

  plsc.subcore_barrier()
  pltpu.sync_copy(acc_sh.at[rows], out_hbm.at[c, rows])


_deg_sc = pl.kernel(
    _deg_body,
    out_type=jax.ShapeDtypeStruct((NC, NPAD), _f32),
    mesh=_sc_mesh,
    compiler_params=_sc_params,
    scratch_types=[
        pltpu.VMEM_SHARED((NPAD,), _f32),
        pltpu.VMEM((KB,), jnp.int32),
        pltpu.VMEM((B,), _f32),
        pltpu.VMEM((RPC,), _f32),
        pltpu.SemaphoreType.DMA,
        pltpu.SemaphoreType.DMA,
    ],
)


def _edge_loop(j0, src_1d, dst_1d, hs_sh, acc_sh, mbig, sg, st):
  def row(j):
    return lax.rem(j - j0, 2 * GD)

  def g_issue(j):
    pltpu.async_copy(hs_sh.at[src_1d.at[pl.ds(j * B, B)]], mbig.at[row(j)],
                     sg)

  def g_wait():
    pltpu.make_async_copy(hs_sh.at[src_1d.at[pl.ds(0, B)]], mbig.at[0],
                          sg).wait()

  def s_issue(j):
    pltpu.async_copy(mbig.at[row(j)],
                     acc_sh.at[dst_1d.at[pl.ds(j * B, B)]], st, add=True)

  def s_wait():
    pltpu.make_async_copy(mbig.at[0], acc_sh.at[dst_1d.at[pl.ds(0, B)]],
                          st).wait()

  def pro(i, carry):
    g_issue(j0 + i)
    return carry
  lax.fori_loop(0, GD, pro, 0)

  def l1(j, carry):
    g_wait()
    s_issue(j)
    g_issue(j + GD)
    return carry
  lax.fori_loop(j0, j0 + GD, l1, 0)

  def l2(j, carry):
    g_wait()
    s_issue(j)
    s_wait()
    g_issue(j + GD)
    return carry
  lax.fori_loop(j0 + GD, K - GD, l2, 0)

  def l3(j, carry):
    g_wait()
    s_issue(j)
    s_wait()
    return carry
  lax.fori_loop(K - GD, K, l3, 0)

  def l4(i, carry):
    s_wait()
    return carry
  lax.fori_loop(0, GD, l4, 0)


def _agg1_body(hs_hbm, ei_hbm, out_hbm,
               acc_sh, hs_sh, src_1d, dst_1d, mbig, zbuf, s0, s1, sg, st):
  c = lax.axis_index("c")
  s = lax.axis_index("s")
  gbase, j0 = _worker_span(c, s)
  rows = pl.ds(s * RPC, RPC)
  pltpu.async_copy(hs_hbm.at[rows, pl.ds(0, D_HID)], hs_sh.at[rows], s0)
  pltpu.async_copy(ei_hbm.at[0, pl.ds(gbase * B, KB)], src_1d, s1)
  pltpu.async_copy(ei_hbm.at[1, pl.ds(gbase * B, KB)], dst_1d, s1)
  _fill_rows(zbuf, RPC, 0.0)
  pltpu.sync_copy(zbuf, acc_sh.at[rows])
  pltpu.make_async_copy(hs_hbm.at[rows, pl.ds(0, D_HID)], hs_sh.at[rows],
                        s0).wait()
  pltpu.make_async_copy(ei_hbm.at[0, pl.ds(0, KB)], src_1d, s1).wait()
  pltpu.make_async_copy(ei_hbm.at[0, pl.ds(0, KB)], dst_1d, s1).wait()
  plsc.subcore_barrier()
  _edge_loop(j0, src_1d, dst_1d, hs_sh, acc_sh, mbig, sg, st)
  plsc.subcore_barrier()
  pltpu.sync_copy(acc_sh.at[rows], out_hbm.at[c, rows])


_agg1_sc = pl.kernel(
    _agg1_body,
    out_type=jax.ShapeDtypeStruct((NC, NPAD, D_HID), _f32),
    mesh=_sc_mesh,
    compiler_params=_sc_params,
    scratch_types=[
        pltpu.VMEM_SHARED((NPAD, D_HID), _f32),
        pltpu.VMEM_SHARED((NPAD, D_HID), _f32),
        pltpu.VMEM((KB,), jnp.int32),
        pltpu.VMEM((KB,), jnp.int32),
        pltpu.VMEM((2 * GD, B, D_HID), _f32),
        pltpu.VMEM((RPC, D_HID), _f32),
        pltpu.SemaphoreType.DMA,
        pltpu.SemaphoreType.DMA,
        pltpu.SemaphoreType.DMA,
        pltpu.SemaphoreType.DMA,
    ],
)


def _agg2_body(aggp_hbm, hs1_hbm, dis_hbm, ei_hbm, out_hbm, hs2_hbm,
               acc_sh, hs_sh, src_1d, dst_1d, mbig,
               p0b, p1b, h1b, disb1, zbuf, zbufz, s0, s1, sg, st):
  c = lax.axis_index("c")
  s = lax.axis_index("s")
  gbase, j0 = _worker_span(c, s)
  rows = pl.ds(s * RPC, RPC)
  pltpu.async_copy(aggp_hbm.at[0, rows], p0b, s0)
  pltpu.async_copy(aggp_hbm.at[1, rows], p1b, s0)
  pltpu.async_copy(hs1_hbm.at[rows, pl.ds(0, D_HID)], h1b, s0)
  pltpu.async_copy(dis_hbm.at[rows], disb1.at[pl.ds(0, RPC)], s0)
  pltpu.async_copy(ei_hbm.at[0, pl.ds(gbase * B, KB)], src_1d, s1)
  pltpu.async_copy(ei_hbm.at[1, pl.ds(gbase * B, KB)], dst_1d, s1)
  _fill_rows(zbufz, RPC, 0.0)
  pltpu.sync_copy(zbufz, acc_sh.at[rows])
  pltpu.make_async_copy(aggp_hbm.at[0, rows], p0b, s0).wait()
  pltpu.make_async_copy(aggp_hbm.at[1, rows], p1b, s0).wait()
  pltpu.make_async_copy(hs1_hbm.at[rows, pl.ds(0, D_HID)], h1b, s0).wait()
  pltpu.make_async_copy(dis_hbm.at[rows], disb1.at[pl.ds(0, RPC)], s0).wait()

  def ew(i, carry):
    a = p0b[i] + p1b[i] + h1b[i]
    d = disb1[pl.ds(i, D_HID)][0]
    zbuf[i] = jnp.maximum(a * d, 0.0) * d
    return carry
  lax.fori_loop(0, RPC, ew, 0)

  pltpu.async_copy(zbuf, hs_sh.at[rows], sg)

  @pl.when(c == 0)
  def _():
    pltpu.async_copy(zbuf, hs2_hbm.at[rows], s0)

  pltpu.make_async_copy(ei_hbm.at[0, pl.ds(0, KB)], src_1d, s1).wait()
  pltpu.make_async_copy(ei_hbm.at[0, pl.ds(0, KB)], dst_1d, s1).wait()
  pltpu.make_async_copy(zbuf, hs_sh.at[rows], sg).wait()
  plsc.subcore_barrier()
  _edge_loop(j0, src_1d, dst_1d, hs_sh, acc_sh, mbig, sg, st)

  @pl.when(c == 0)
  def _():
    pltpu.make_async_copy(zbuf, hs2_hbm.at[rows], s0).wait()

  plsc.subcore_barrier()
  pltpu.sync_copy(acc_sh.at[rows], out_hbm.at[c, rows])


_agg2_sc = pl.kernel(
    _agg2_body,
    out_type=(jax.ShapeDtypeStruct((NC, NPAD, D_HID), _f32),
              jax.ShapeDtypeStruct((NPAD, D_HID), _f32)),
    mesh=_sc_mesh,
    compiler_params=_sc_params,
    scratch_types=[
        pltpu.VMEM_SHARED((NPAD, D_HID), _f32),
        pltpu.VMEM_SHARED((NPAD, D_HID), _f32),
        pltpu.VMEM((KB,), jnp.int32),
        pltpu.VMEM((KB,), jnp.int32),
        pltpu.VMEM((2 * GD, B, D_HID), _f32),
        pltpu.VMEM((RPC, D_HID), _f32),
        pltpu.VMEM((RPC, D_HID), _f32),
        pltpu.VMEM((RPC, D_HID), _f32),
        pltpu.VMEM((RPC + D_HID,), _f32),
        pltpu.VMEM((RPC, D_HID), _f32),
        pltpu.VMEM((RPC, D_HID), _f32),
        pltpu.SemaphoreType.DMA,
        pltpu.SemaphoreType.DMA,
        pltpu.SemaphoreType.DMA,
        pltpu.SemaphoreType.DMA,
    ],
)


def _mm_body(x_ref, w1_ref, xw_ref):
  xw_ref[...] = jnp.dot(x_ref[...], w1_ref[...], preferred_element_type=_f32)


def _scale_body(xw_ref, degp_ref, hs1_ref, dis_ref):
  degp = degp_ref[...]
  deg = degp[0] + degp[1] + 1.0
  dis = lax.rsqrt(deg)
  dis_ref[...] = dis
  hs1_ref[:N, :D_HID] = xw_ref[...] * dis[:N, None]


def _fin_body(aggp_ref, hs2_ref, dis_ref, w2_ref, out_ref):
  aggp = aggp_ref[...]
  dis = dis_ref[...]
  a = (aggp[0, :N, :] + aggp[1, :N, :] + hs2_ref[:N, :]) * dis[:N, None]
  out_ref[...] = jnp.dot(a, w2_ref[...], preferred_element_type=_f32)


def kernel(x, edge_index, W1, W2):
  xw = pl.pallas_call(
      _mm_body,
      out_shape=jax.ShapeDtypeStruct((N, D_HID), _f32),
  )(x, W1)

  degp = _deg_sc(edge_index)

  hs1, dis = pl.pallas_call(
      _scale_body,
      out_shape=(jax.ShapeDtypeStruct((NPAD, D_IN), _f32),
                 jax.ShapeDtypeStruct((NPAD,), _f32)),
  )(xw, degp)

  agg1 = _agg1_sc(hs1, edge_index)
  agg2, hs2 = _agg2_sc(agg1, hs1, dis, edge_index)

  out = pl.pallas_call(
      _fin_body,
      out_shape=jax.ShapeDtypeStruct((N, D_OUT), _f32),
  )(agg2, hs2, dis, W2)

  return out


# speedup vs baseline: 1.0065x; 1.0065x over previous
import jax
import jax.numpy as jnp
from jax import lax
from jax.experimental import pallas as pl
from jax.experimental.pallas import tpu as pltpu
from jax.experimental.pallas import tpu_sc as plsc

N = 10000
E = 320000
D_IN = 128
D_HID = 16
D_OUT = 128

NC = 2
NS = 16
NW = NC * NS
B = 128
GTOT = E // B
K = 79
KB = K * B
NPAD = 10240
RPC = NPAD // NS
DEPTH = 4
GD = 8

_sc_mesh = plsc.VectorSubcoreMesh(
    core_axis_name="c", subcore_axis_name="s", num_cores=NC, num_subcores=NS)
_sc_params = pltpu.CompilerParams(use_tc_tiling_on_sc=False)

_f32 = jnp.float32


def _worker_span(c, s):
  wid = s * NC + c
  gbase = jnp.minimum(wid * K, GTOT - K)
  j0 = wid * K - gbase
  return gbase, j0


def _fill_rows(buf, n, val):
  def st(i, carry):
    buf[i] = jnp.full((D_HID,), val, _f32)
    return carry
  lax.fori_loop(0, n, st, 0)


def _fill_flat(buf, n, val):
  def st(i, carry):
    buf[pl.ds(i * D_HID, D_HID)] = jnp.full((D_HID,), val, _f32)
    return carry
  lax.fori_loop(0, n // D_HID, st, 0)


def _deg_body(ei_hbm, out_hbm, acc_sh, dst_1d, ones_v, zbuf, sem0, sem1):
  c = lax.axis_index("c")
  s = lax.axis_index("s")
  gbase, j0 = _worker_span(c, s)
  rows = pl.ds(s * RPC, RPC)
  pltpu.async_copy(ei_hbm.at[1, pl.ds(gbase * B, KB)], dst_1d, sem1)
  _fill_flat(zbuf, RPC, 0.0)
  _fill_flat(ones_v, B, 1.0)
  pltpu.sync_copy(zbuf, acc_sh.at[rows])
  pltpu.make_async_copy(ei_hbm.at[1, pl.ds(0, KB)], dst_1d, sem1).wait()
  plsc.subcore_barrier()

  def issue(j):
    pltpu.async_copy(ones_v, acc_sh.at[dst_1d.at[pl.ds(j * B, B)]], sem0,
                     add=True)

  def wait_one():
    pltpu.make_async_copy(ones_v, acc_sh.at[dst_1d.at[pl.ds(0, B)]],
                          sem0).wait()

  def prime(j, carry):
    issue(j0 + j)
    return carry
  lax.fori_loop(0, DEPTH, prime, 0)

  def step(j, carry):
    wait_one()
    issue(j)
    return carry
  lax.fori_loop(j0 + DEPTH, K, step, 0)

  def drain(j, carry):
    wait_one()
    return carry
  lax.fori_loop(0, DEPTH, drain, 0)

  plsc.subcore_barrier()
  pltpu.sync_copy(acc_sh.at[rows], zbuf)
  pltpu.sync_copy(zbuf, out_hbm.at[c, rows])


_deg_sc = pl.kernel(
    _deg_body,
    out_type=jax.ShapeDtypeStruct((NC, NPAD), _f32),
    mesh=_sc_mesh,
    compiler_params=_sc_params,
    scratch_types=[
        pltpu.VMEM_SHARED((NPAD,), _f32),
        pltpu.VMEM((KB,), jnp.int32),
        pltpu.VMEM((B,), _f32),
        pltpu.VMEM((RPC,), _f32),
        pltpu.SemaphoreType.DMA,
        pltpu.SemaphoreType.DMA,
    ],
)


def _edge_loop(j0, src_1d, dst_1d, hs_sh, acc_sh, mbig, sg, st):
  def row(j):
    return lax.rem(j - j0, 2 * GD)

  def g_issue(j):
    pltpu.async_copy(hs_sh.at[src_1d.at[pl.ds(j * B, B)]], mbig.at[row(j)],
                     sg)

  def g_wait():
    pltpu.make_async_copy(hs_sh.at[src_1d.at[pl.ds(0, B)]], mbig.at[0],
                          sg).wait()

  def s_issue(j):
    pltpu.async_copy(mbig.at[row(j)],
                     acc_sh.at[dst_1d.at[pl.ds(j * B, B)]], st, add=True)

  def s_wait():
    pltpu.make_async_copy(mbig.at[0], acc_sh.at[dst_1d.at[pl.ds(0, B)]],
                          st).wait()

  def pro(i, carry):
    g_issue(j0 + i)
    return carry
  lax.fori_loop(0, GD, pro, 0)

  def l1(j, carry):
    g_wait()
    s_issue(j)
    g_issue(j + GD)
    return carry
  lax.fori_loop(j0, j0 + GD, l1, 0)

  def l2(j, carry):
    g_wait()
    s_issue(j)
    s_wait()
    g_issue(j + GD)
    return carry
  lax.fori_loop(j0 + GD, K - GD, l2, 0)

  def l3(j, carry):
    g_wait()
    s_issue(j)
    s_wait()
    return carry
  lax.fori_loop(K - GD, K, l3, 0)

  def l4(i, carry):
    s_wait()
    return carry
  lax.fori_loop(0, GD, l4, 0)


def _agg1_body(hs_hbm, ei_hbm, out_hbm,
               acc_sh, hs_sh, src_1d, dst_1d, mbig, zbuf, s0, s1, sg, st):
  c = lax.axis_index("c")
  s = lax.axis_index("s")
  gbase, j0 = _worker_span(c, s)
  rows = pl.ds(s * RPC, RPC)
  pltpu.async_copy(hs_hbm.at[rows, pl.ds(0, D_HID)], hs_sh.at[rows], s0)
  pltpu.async_copy(ei_hbm.at[0, pl.ds(gbase * B, KB)], src_1d, s1)
  pltpu.async_copy(ei_hbm.at[1, pl.ds(gbase * B, KB)], dst_1d, s1)
  _fill_rows(zbuf, RPC, 0.0)
  pltpu.sync_copy(zbuf, acc_sh.at[rows])
  pltpu.make_async_copy(hs_hbm.at[rows, pl.ds(0, D_HID)], hs_sh.at[rows],
                        s0).wait()
  pltpu.make_async_copy(ei_hbm.at[0, pl.ds(0, KB)], src_1d, s1).wait()
  pltpu.make_async_copy(ei_hbm.at[0, pl.ds(0, KB)], dst_1d, s1).wait()
  plsc.subcore_barrier()
  _edge_loop(j0, src_1d, dst_1d, hs_sh, acc_sh, mbig, sg, st)
  plsc.subcore_barrier()
  pltpu.sync_copy(acc_sh.at[rows], zbuf)
  pltpu.sync_copy(zbuf, out_hbm.at[c, rows])


_agg1_sc = pl.kernel(
    _agg1_body,
    out_type=jax.ShapeDtypeStruct((NC, NPAD, D_HID), _f32),
    mesh=_sc_mesh,
    compiler_params=_sc_params,
    scratch_types=[
        pltpu.VMEM_SHARED((NPAD, D_HID), _f32),
        pltpu.VMEM_SHARED((NPAD, D_HID), _f32),
        pltpu.VMEM((KB,), jnp.int32),
        pltpu.VMEM((KB,), jnp.int32),
        pltpu.VMEM((2 * GD, B, D_HID), _f32),
        pltpu.VMEM((RPC, D_HID), _f32),
        pltpu.SemaphoreType.DMA,
        pltpu.SemaphoreType.DMA,
        pltpu.SemaphoreType.DMA,
        pltpu.SemaphoreType.DMA,
    ],
)


def _agg2_body(aggp_hbm, hs1_hbm, dis_hbm, ei_hbm, out_hbm, hs2_hbm,
               acc_sh, hs_sh, src_1d, dst_1d, mbig,
               p0b, p1b, h1b, disb1, zbuf, zbufz, s0, s1, sg, st):
  c = lax.axis_index("c")
  s = lax.axis_index("s")
  gbase, j0 = _worker_span(c, s)
  rows = pl.ds(s * RPC, RPC)
  pltpu.async_copy(aggp_hbm.at[0, rows], p0b, s0)
  pltpu.async_copy(aggp_hbm.at[1, rows], p1b, s0)
  pltpu.async_copy(hs1_hbm.at[rows, pl.ds(0, D_HID)], h1b, s0)
  pltpu.async_copy(dis_hbm.at[rows], disb1.at[pl.ds(0, RPC)], s0)
  pltpu.async_copy(ei_hbm.at[0, pl.ds(gbase * B, KB)], src_1d, s1)
  pltpu.async_copy(ei_hbm.at[1, pl.ds(gbase * B, KB)], dst_1d, s1)
  _fill_rows(zbufz, RPC, 0.0)
  pltpu.sync_copy(zbufz, acc_sh.at[rows])
  pltpu.make_async_copy(aggp_hbm.at[0, rows], p0b, s0).wait()
  pltpu.make_async_copy(aggp_hbm.at[1, rows], p1b, s0).wait()
  pltpu.make_async_copy(hs1_hbm.at[rows, pl.ds(0, D_HID)], h1b, s0).wait()
  pltpu.make_async_copy(dis_hbm.at[rows], disb1.at[pl.ds(0, RPC)], s0).wait()

  def ew(i, carry):
    a = p0b[i] + p1b[i] + h1b[i]
    d = disb1[pl.ds(i, D_HID)][0]
    zbuf[i] = jnp.maximum(a * d, 0.0) * d
    return carry
  lax.fori_loop(0, RPC, ew, 0)

  pltpu.async_copy(zbuf, hs_sh.at[rows], sg)

  @pl.when(c == 0)
  def _():
    pltpu.async_copy(zbuf, hs2_hbm.at[rows], s0)

  pltpu.make_async_copy(ei_hbm.at[0, pl.ds(0, KB)], src_1d, s1).wait()
  pltpu.make_async_copy(ei_hbm.at[0, pl.ds(0, KB)], dst_1d, s1).wait()
  pltpu.make_async_copy(zbuf, hs_sh.at[rows], sg).wait()
  plsc.subcore_barrier()
  _edge_loop(j0, src_1d, dst_1d, hs_sh, acc_sh, mbig, sg, st)

  @pl.when(c == 0)
  def _():
    pltpu.make_async_copy(zbuf, hs2_hbm.at[rows], s0).wait()

  plsc.subcore_barrier()
  pltpu.sync_copy(acc_sh.at[rows], zbuf)
  pltpu.sync_copy(zbuf, out_hbm.at[c, rows])


_agg2_sc = pl.kernel(
    _agg2_body,
    out_type=(jax.ShapeDtypeStruct((NC, NPAD, D_HID), _f32),
              jax.ShapeDtypeStruct((NPAD, D_HID), _f32)),
    mesh=_sc_mesh,
    compiler_params=_sc_params,
    scratch_types=[
        pltpu.VMEM_SHARED((NPAD, D_HID), _f32),
        pltpu.VMEM_SHARED((NPAD, D_HID), _f32),
        pltpu.VMEM((KB,), jnp.int32),
        pltpu.VMEM((KB,), jnp.int32),
        pltpu.VMEM((2 * GD, B, D_HID), _f32),
        pltpu.VMEM((RPC, D_HID), _f32),
        pltpu.VMEM((RPC, D_HID), _f32),
        pltpu.VMEM((RPC, D_HID), _f32),
        pltpu.VMEM((RPC + D_HID,), _f32),
        pltpu.VMEM((RPC, D_HID), _f32),
        pltpu.VMEM((RPC, D_HID), _f32),
        pltpu.SemaphoreType.DMA,
        pltpu.SemaphoreType.DMA,
        pltpu.SemaphoreType.DMA,
        pltpu.SemaphoreType.DMA,
    ],
)


def _mm_body(x_ref, w1_ref, xw_ref):
  xw_ref[...] = jnp.dot(x_ref[...], w1_ref[...], preferred_element_type=_f32)


def _scale_body(xw_ref, degp_ref, hs1_ref, dis_ref):
  degp = degp_ref[...]
  deg = degp[0] + degp[1] + 1.0
  dis = lax.rsqrt(deg)
  dis_ref[...] = dis
  hs1_ref[:N, :D_HID] = xw_ref[...] * dis[:N, None]


def _fin_body(aggp_ref, hs2_ref, dis_ref, w2_ref, out_ref):
  aggp = aggp_ref[...]
  dis = dis_ref[...]
  a = (aggp[0, :N, :] + aggp[1, :N, :] + hs2_ref[:N, :]) * dis[:N, None]
  out_ref[...] = jnp.dot(a, w2_ref[...], preferred_element_type=_f32)


def kernel(x, edge_index, W1, W2):
  xw = pl.pallas_call(
      _mm_body,
      out_shape=jax.ShapeDtypeStruct((N, D_HID), _f32),
  )(x, W1)

  degp = _deg_sc(edge_index)

  hs1, dis = pl.pallas_call(
      _scale_body,
      out_shape=(jax.ShapeDtypeStruct((NPAD, D_IN), _f32),
                 jax.ShapeDtypeStruct((NPAD,), _f32)),
  )(xw, degp)

  agg1 = _agg1_sc(hs1, edge_index)
  agg2, hs2 = _agg2_sc(agg1, hs1, dis, edge_index)

  out = pl.pallas_call(
      _fin_body,
      out_shape=jax.ShapeDtypeStruct((N, D_OUT), _f32),
  )(agg2, hs2, dis, W2)

  return out
